# baseline (speedup 1.0000x reference)
import functools
import jax
import jax.numpy as jnp
from jax import lax
from jax.experimental import pallas as pl
from jax.experimental.pallas import tpu as pltpu
from jax.experimental.pallas import tpu_sc as plsc

N, E, D_FEAT, D_EDGE, H, OUT, G = 10000, 320000, 128, 16, 32, 32, 64
NP = 10240
W = 48
NC, NS = 2, 16
NWORK = NC * NS
CH = 128
EP = 327680
CPW = EP // (NWORK * CH)
CR = CH // 4
RPW = NP // NS
ZR = 160


def _node_body(x, nw1, nb1, nw2, nb2, w1d, w1s, cb1, h_o, a_o, b_o):
    hid = jnp.maximum(jnp.dot(x[...], nw1[...],
                              preferred_element_type=jnp.float32) + nb1[...], 0.0)
    h = jnp.dot(hid, nw2[...], preferred_element_type=jnp.float32) + nb2[...]
    h_o[...] = h
    a_o[...] = jnp.dot(h, w1d[...], preferred_element_type=jnp.float32) + cb1[...]
    b_o[...] = jnp.dot(h, w1s[...], preferred_element_type=jnp.float32)


def _edge_body(attr, ew1, eb1, ew2, eb2, w1e, c_o):
    a = attr[...]
    for j in range(8):
        sl = a[:, D_EDGE * j:D_EDGE * (j + 1)]
        hid = jnp.maximum(jnp.dot(sl, ew1[...],
                                  preferred_element_type=jnp.float32) + eb1[...], 0.0)
        ea = jnp.dot(hid, ew2[...], preferred_element_type=jnp.float32) + eb2[...]
        c_o[:, H * j:H * (j + 1)] = jnp.dot(ea, w1e[...],
                                            preferred_element_type=jnp.float32)


def _update_body(sp, h, w2e, w1d, w1s, cb1, a_o, b_o):
    s = sp[0] + sp[1]
    agg = jnp.dot(s, w2e[...], preferred_element_type=jnp.float32)
    h2 = jnp.maximum(h[...] + agg, 0.0)
    a_o[...] = jnp.dot(h2, w1d[...], preferred_element_type=jnp.float32) + cb1[...]
    b_o[...] = jnp.dot(h2, w1s[...], preferred_element_type=jnp.float32)


def _pool_body(sp, w2e, batch, out):
    i = pl.program_id(0)
    s = sp[0] + sp[1]
    node_out = jnp.dot(s, w2e[...], preferred_element_type=jnp.float32)
    rows = s.shape[0]
    rid = lax.broadcasted_iota(jnp.int32, (rows, 1), 0) + i * rows
    node_out = jnp.where(rid < N, node_out, 0.0)
    b_ids = batch[0]
    gid = lax.broadcasted_iota(jnp.int32, (G, rows), 0)
    col = lax.broadcasted_iota(jnp.int32, (G, rows), 1) + i * rows
    oh = jnp.where((gid == b_ids) & (col < N), 1.0, 0.0).astype(jnp.float32)
    acc = jnp.dot(oh, node_out, preferred_element_type=jnp.float32)

    @pl.when(i == 0)
    def _():
        out[...] = jnp.zeros_like(out)

    out[...] += acc


_BN = 512
_BE8 = 512


def _full(shape):
    return pl.BlockSpec(shape, lambda i: tuple(0 for _ in shape))


def _tc_node(x_p, nw1, nb1, nw2, nb2, w1d, w1s, cb1):
    grid = NP // _BN
    outs = [jax.ShapeDtypeStruct((NP, H), jnp.float32)] * 3
    return pl.pallas_call(
        _node_body,
        grid=(grid,),
        in_specs=[
            pl.BlockSpec((_BN, D_FEAT), lambda i: (i, 0)),
            _full((D_FEAT, H)), _full((1, H)), _full((H, H)), _full((1, H)),
            _full((H, H)), _full((H, H)), _full((1, H)),
        ],
        out_specs=[pl.BlockSpec((_BN, H), lambda i: (i, 0))] * 3,
        out_shape=outs,
    )(x_p, nw1, nb1, nw2, nb2, w1d, w1s, cb1)


def _tc_edge(attr8, ew1, eb1, ew2, eb2, w1e):
    grid = (EP // 8) // _BE8
    return pl.pallas_call(
        _edge_body,
        grid=(grid,),
        in_specs=[
            pl.BlockSpec((_BE8, 8 * D_EDGE),
                         lambda i: (jnp.minimum(i, (E // 8) // _BE8), 0)),
            _full((D_EDGE, H)), _full((1, H)), _full((H, H)), _full((1, H)),
            _full((H, H)),
        ],
        out_specs=pl.BlockSpec((_BE8, 8 * H), lambda i: (i, 0)),
        out_shape=jax.ShapeDtypeStruct((EP // 8, 8 * H), jnp.float32),
    )(attr8, ew1, eb1, ew2, eb2, w1e)


def _tc_update(sp, h, w2e, w1d, w1s, cb1):
    grid = NP // _BN
    outs = [jax.ShapeDtypeStruct((NP, H), jnp.float32)] * 2
    return pl.pallas_call(
        _update_body,
        grid=(grid,),
        in_specs=[
            pl.BlockSpec((NC, _BN, W), lambda i: (0, i, 0)),
            pl.BlockSpec((_BN, H), lambda i: (i, 0)),
            _full((W, H)), _full((H, H)), _full((H, H)), _full((1, H)),
        ],
        out_specs=[pl.BlockSpec((_BN, H), lambda i: (i, 0))] * 2,
        out_shape=outs,
    )(sp, h, w2e, w1d, w1s, cb1)


def _tc_pool(sp, w2e, batch3d):
    grid = NP // _BN
    return pl.pallas_call(
        _pool_body,
        grid=(grid,),
        in_specs=[
            pl.BlockSpec((NC, _BN, W), lambda i: (0, i, 0)),
            _full((W, OUT)),
            pl.BlockSpec((1, 1, _BN), lambda i: (i, 0, 0)),
        ],
        out_specs=pl.BlockSpec((G, OUT), lambda i: (0, 0)),
        out_shape=jax.ShapeDtypeStruct((G, OUT), jnp.float32),
    )(sp, w2e, batch3d)


def _sc_body(a_t, b_t, c4, src2, dst2, out_h,
             sidx, didx, a0, a1, b0, b1, c0, c1, t0, t1, z_v, s_sh,
             sa0, sa1, sb0, sb1, sc0, sc1, ss0, ss1):
    cid = lax.axis_index("c")
    sid = lax.axis_index("s")
    wid = sid * NC + cid
    r0 = sid * RPW
    wrow0 = wid * CPW

    zv = jnp.zeros((16,), jnp.float32)

    def _zrow(i, c):
        z_v[i, pl.ds(0, 16)] = zv
        z_v[i, pl.ds(16, 16)] = zv
        z_v[i, pl.ds(32, 16)] = zv
        return c

    lax.fori_loop(0, ZR, _zrow, 0)

    onev = jnp.where(lax.iota(jnp.int32, 16) == 0,
                     jnp.float32(1.0), jnp.float32(0.0))

    def _trow(i, c):
        t0[i, pl.ds(32, 16)] = onev
        t1[i, pl.ds(32, 16)] = onev
        return c

    lax.fori_loop(0, CH, _trow, 0)

    pltpu.sync_copy(src2.at[pl.ds(wrow0, CPW)], sidx)
    pltpu.sync_copy(dst2.at[pl.ds(wrow0, CPW)], didx)

    for q in range(RPW // ZR):
        pltpu.sync_copy(z_v, s_sh.at[pl.ds(r0 + q * ZR, ZR)])
    plsc.subcore_barrier()

    bufs = ((a0, b0, c0, t0, sa0, sb0, sc0, ss0),
            (a1, b1, c1, t1, sa1, sb1, sc1, ss1))

    def _issue(g, bu):
        aV, bV, cV, _, sa, sb, sc, _ = bu
        pltpu.async_copy(c4.at[pl.ds((wrow0 + g) * (CH * H), CH * H)], cV, sc)
        pltpu.async_copy(a_t.at[didx.at[g]], aV, sa)
        pltpu.async_copy(b_t.at[sidx.at[g]], bV, sb)

    def _wait_loads(bu):
        aV, bV, cV, _, sa, sb, sc, _ = bu
        pltpu.make_async_copy(c4.at[pl.ds(0, CH * H)], cV, sc).wait()
        pltpu.make_async_copy(a_t.at[didx.at[0]], aV, sa).wait()
        pltpu.make_async_copy(b_t.at[sidx.at[0]], bV, sb).wait()

    def _compute(bu):
        aV, bV, cV, tV = bu[0], bu[1], bu[2], bu[3]

        @plsc.parallel_loop(0, CH, step=1, unroll=4)
        def _(e):
            v0 = aV[e, pl.ds(0, 16)] + bV[e, pl.ds(0, 16)] + cV[pl.ds(H * e, 16)]
            tV[e, pl.ds(0, 16)] = jnp.maximum(v0, 0.0)
            v1 = (aV[e, pl.ds(16, 16)] + bV[e, pl.ds(16, 16)]
                  + cV[pl.ds(H * e + 16, 16)])
            tV[e, pl.ds(16, 16)] = jnp.maximum(v1, 0.0)

    def _scatter(g, bu):
        tV, ss = bu[3], bu[7]
        pltpu.async_copy(tV, s_sh.at[didx.at[g]], ss, add=True)

    def _wait_scatter(bu):
        tV, ss = bu[3], bu[7]
        pltpu.make_async_copy(tV, s_sh.at[didx.at[0]], ss).wait()

    _issue(0, bufs[0])

    def _step(k, c):
        g0 = 2 * k
        _issue(g0 + 1, bufs[1])
        _wait_loads(bufs[0])

        @pl.when(k > 0)
        def _():
            _wait_scatter(bufs[0])

        _compute(bufs[0])
        _scatter(g0, bufs[0])

        @pl.when(k < CPW // 2 - 1)
        def _():
            _issue(g0 + 2, bufs[0])

        _wait_loads(bufs[1])

        @pl.when(k > 0)
        def _():
            _wait_scatter(bufs[1])

        _compute(bufs[1])
        _scatter(g0 + 1, bufs[1])
        return c

    lax.fori_loop(0, CPW // 2, _step, 0)
    _wait_scatter(bufs[0])
    _wait_scatter(bufs[1])

    plsc.subcore_barrier()
    pltpu.sync_copy(s_sh.at[pl.ds(r0, RPW)], out_h.at[cid, pl.ds(r0, RPW)])


@functools.cache
def _sc_pass():
    return pl.kernel(
        _sc_body,
        out_type=jax.ShapeDtypeStruct((NC, NP, W), jnp.float32),
        mesh=plsc.VectorSubcoreMesh(core_axis_name="c", subcore_axis_name="s"),
        compiler_params=pltpu.CompilerParams(use_tc_tiling_on_sc=False),
        scratch_types=(
            [pltpu.VMEM((CPW, CH), jnp.int32)] * 2
            + [pltpu.VMEM((CH, H), jnp.float32)] * 4
            + [pltpu.VMEM((CH * H,), jnp.float32)] * 2
            + [pltpu.VMEM((CH, W), jnp.float32)] * 2
            + [pltpu.VMEM((ZR, W), jnp.float32)]
            + [pltpu.VMEM_SHARED((NP, W), jnp.float32)]
            + [pltpu.SemaphoreType.DMA] * 8
        ),
    )


def kernel(x, edge_attr, edge_index, batch,
           node_W1, node_b1, node_W2, node_b2,
           edge_W1, edge_b1, edge_W2, edge_b2,
           c1_W1, c1_b1, c1_W2, c1_b2,
           c2_W1, c2_b1, c2_W2, c2_b2):
    f32 = jnp.float32
    x_p = jnp.pad(x, ((0, NP - N), (0, 0)))
    batch3d = jnp.pad(batch, (0, NP - N)).reshape(NP // _BN, 1, _BN)
    src2 = jnp.pad(edge_index[0], (0, EP - E),
                   constant_values=NP - 1).reshape(EP // CH, CH)
    dst2 = jnp.pad(edge_index[1], (0, EP - E),
                   constant_values=NP - 1).reshape(EP // CH, CH)

    def row(b):
        return b.reshape(1, -1).astype(f32)

    def w2ext(w2, b2, cols):
        return jnp.concatenate(
            [w2, b2.reshape(1, cols), jnp.zeros((W - H - 1, cols), f32)], axis=0)

    w2e1 = w2ext(c1_W2, c1_b2, H)
    w2e2 = w2ext(c2_W2, c2_b2, OUT)

    h, a1, b1t = _tc_node(x_p, node_W1, row(node_b1), node_W2, row(node_b2),
                          c1_W1[0:H], c1_W1[H:2 * H], row(c1_b1))
    eb1r, eb2r = row(edge_b1), row(edge_b2)
    attr8 = edge_attr.reshape(E // 8, 8 * D_EDGE)
    c1t = _tc_edge(attr8, edge_W1, eb1r, edge_W2, eb2r, c1_W1[2 * H:3 * H])

    sc = _sc_pass()
    sp1 = sc(a1, b1t, c1t.reshape(-1), src2, dst2)
    c2t = _tc_edge(attr8, edge_W1, eb1r, edge_W2, eb2r, c2_W1[2 * H:3 * H])
    a2, b2t = _tc_update(sp1, h, w2e1, c2_W1[0:H], c2_W1[H:2 * H], row(c2_b1))
    sp2 = sc(a2, b2t, c2t.reshape(-1), src2, dst2)
    return _tc_pool(sp2, w2e2, batch3d)


# speedup vs baseline: 1.1500x; 1.1500x over previous
import functools
import jax
import jax.numpy as jnp
from jax import lax
from jax.experimental import pallas as pl
from jax.experimental.pallas import tpu as pltpu
from jax.experimental.pallas import tpu_sc as plsc

N, E, D_FEAT, D_EDGE, H, OUT, G = 10000, 320000, 128, 16, 32, 32, 64
NP = 10240
W = 48
NC, NS = 2, 16
NWORK = NC * NS
CH = 128
EP = 327680
CPW = EP // (NWORK * CH)
CR = CH // 4
RPW = NP // NS
ZR = 160


def _node_body(x, nw1, nb1, nw2, nb2, w1d, w1s, cb1, h_o, a_o, b_o):
    hid = jnp.maximum(jnp.dot(x[...], nw1[...],
                              preferred_element_type=jnp.float32) + nb1[...], 0.0)
    h = jnp.dot(hid, nw2[...], preferred_element_type=jnp.float32) + nb2[...]
    h_o[...] = h
    a_o[...] = jnp.dot(h, w1d[...], preferred_element_type=jnp.float32) + cb1[...]
    b_o[...] = jnp.dot(h, w1s[...], preferred_element_type=jnp.float32)


def _edge_body(attr4, ew1b, eb1b, m1b, r1b, m2b, r2b, c1_o, c2_o):
    hid = jnp.maximum(jnp.dot(attr4[...], ew1b[...],
                              preferred_element_type=jnp.float32) + eb1b[...], 0.0)
    c1_o[...] = jnp.dot(hid, m1b[...], preferred_element_type=jnp.float32) + r1b[...]
    c2_o[...] = jnp.dot(hid, m2b[...], preferred_element_type=jnp.float32) + r2b[...]


def _update_body(sp, h, w2e, w1d, w1s, cb1, a_o, b_o):
    s = sp[0] + sp[1]
    agg = jnp.dot(s, w2e[...], preferred_element_type=jnp.float32)
    h2 = jnp.maximum(h[...] + agg, 0.0)
    a_o[...] = jnp.dot(h2, w1d[...], preferred_element_type=jnp.float32) + cb1[...]
    b_o[...] = jnp.dot(h2, w1s[...], preferred_element_type=jnp.float32)


def _pool_body(sp, w2e, batch, out):
    i = pl.program_id(0)
    s = sp[0] + sp[1]
    node_out = jnp.dot(s, w2e[...], preferred_element_type=jnp.float32)
    rows = s.shape[0]
    rid = lax.broadcasted_iota(jnp.int32, (rows, 1), 0) + i * rows
    node_out = jnp.where(rid < N, node_out, 0.0)
    b_ids = batch[0]
    gid = lax.broadcasted_iota(jnp.int32, (G, rows), 0)
    col = lax.broadcasted_iota(jnp.int32, (G, rows), 1) + i * rows
    oh = jnp.where((gid == b_ids) & (col < N), 1.0, 0.0).astype(jnp.float32)
    acc = jnp.dot(oh, node_out, preferred_element_type=jnp.float32)

    @pl.when(i == 0)
    def _():
        out[...] = jnp.zeros_like(out)

    out[...] += acc


_BN = 512
_BE4 = 1024


def _full(shape):
    return pl.BlockSpec(shape, lambda i: tuple(0 for _ in shape))


def _tc_node(x_p, nw1, nb1, nw2, nb2, w1d, w1s, cb1):
    grid = NP // _BN
    outs = [jax.ShapeDtypeStruct((NP, H), jnp.float32)] * 3
    return pl.pallas_call(
        _node_body,
        grid=(grid,),
        in_specs=[
            pl.BlockSpec((_BN, D_FEAT), lambda i: (i, 0)),
            _full((D_FEAT, H)), _full((1, H)), _full((H, H)), _full((1, H)),
            _full((H, H)), _full((H, H)), _full((1, H)),
        ],
        out_specs=[pl.BlockSpec((_BN, H), lambda i: (i, 0))] * 3,
        out_shape=outs,
    )(x_p, nw1, nb1, nw2, nb2, w1d, w1s, cb1)


def _tc_edge(attr4, ew1b, eb1b, m1b, r1b, m2b, r2b):
    grid = (EP // 4) // _BE4
    outs = [jax.ShapeDtypeStruct((EP // 4, 4 * H), jnp.float32)] * 2
    return pl.pallas_call(
        _edge_body,
        grid=(grid,),
        in_specs=[
            pl.BlockSpec((_BE4, 4 * D_EDGE),
                         lambda i: (jnp.minimum(i, (E // 4) // _BE4), 0)),
            _full((4 * D_EDGE, 4 * H)), _full((1, 4 * H)),
            _full((4 * H, 4 * H)), _full((1, 4 * H)),
            _full((4 * H, 4 * H)), _full((1, 4 * H)),
        ],
        out_specs=[pl.BlockSpec((_BE4, 4 * H), lambda i: (i, 0))] * 2,
        out_shape=outs,
    )(attr4, ew1b, eb1b, m1b, r1b, m2b, r2b)


def _tc_update(sp, h, w2e, w1d, w1s, cb1):
    grid = NP // _BN
    outs = [jax.ShapeDtypeStruct((NP, H), jnp.float32)] * 2
    return pl.pallas_call(
        _update_body,
        grid=(grid,),
        in_specs=[
            pl.BlockSpec((NC, _BN, W), lambda i: (0, i, 0)),
            pl.BlockSpec((_BN, H), lambda i: (i, 0)),
            _full((W, H)), _full((H, H)), _full((H, H)), _full((1, H)),
        ],
        out_specs=[pl.BlockSpec((_BN, H), lambda i: (i, 0))] * 2,
        out_shape=outs,
    )(sp, h, w2e, w1d, w1s, cb1)


def _tc_pool(sp, w2e, batch3d):
    grid = NP // _BN
    return pl.pallas_call(
        _pool_body,
        grid=(grid,),
        in_specs=[
            pl.BlockSpec((NC, _BN, W), lambda i: (0, i, 0)),
            _full((W, OUT)),
            pl.BlockSpec((1, 1, _BN), lambda i: (i, 0, 0)),
        ],
        out_specs=pl.BlockSpec((G, OUT), lambda i: (0, 0)),
        out_shape=jax.ShapeDtypeStruct((G, OUT), jnp.float32),
    )(sp, w2e, batch3d)


def _sc_body(a_t, b_t, c4, src2, dst2, out_h,
             sidx, didx, a0, a1, b0, b1, c0, c1, t0, t1, z_v, s_sh,
             sa0, sa1, sb0, sb1, sc0, sc1, ss0, ss1):
    cid = lax.axis_index("c")
    sid = lax.axis_index("s")
    wid = sid * NC + cid
    r0 = sid * RPW
    wrow0 = wid * CPW

    zv = jnp.zeros((16,), jnp.float32)

    def _zrow(i, c):
        z_v[i, pl.ds(0, 16)] = zv
        z_v[i, pl.ds(16, 16)] = zv
        z_v[i, pl.ds(32, 16)] = zv
        return c

    lax.fori_loop(0, ZR, _zrow, 0)

    onev = jnp.where(lax.iota(jnp.int32, 16) == 0,
                     jnp.float32(1.0), jnp.float32(0.0))

    def _trow(i, c):
        t0[i, pl.ds(32, 16)] = onev
        t1[i, pl.ds(32, 16)] = onev
        return c

    lax.fori_loop(0, CH, _trow, 0)

    pltpu.sync_copy(src2.at[pl.ds(wrow0, CPW)], sidx)
    pltpu.sync_copy(dst2.at[pl.ds(wrow0, CPW)], didx)

    for q in range(RPW // ZR):
        pltpu.sync_copy(z_v, s_sh.at[pl.ds(r0 + q * ZR, ZR)])
    plsc.subcore_barrier()

    bufs = ((a0, b0, c0, t0, sa0, sb0, sc0, ss0),
            (a1, b1, c1, t1, sa1, sb1, sc1, ss1))

    def _issue(g, bu):
        aV, bV, cV, _, sa, sb, sc, _ = bu
        pltpu.async_copy(c4.at[pl.ds((wrow0 + g) * (CH * H), CH * H)], cV, sc)
        pltpu.async_copy(a_t.at[didx.at[g]], aV, sa)
        pltpu.async_copy(b_t.at[sidx.at[g]], bV, sb)

    def _wait_loads(bu):
        aV, bV, cV, _, sa, sb, sc, _ = bu
        pltpu.make_async_copy(c4.at[pl.ds(0, CH * H)], cV, sc).wait()
        pltpu.make_async_copy(a_t.at[didx.at[0]], aV, sa).wait()
        pltpu.make_async_copy(b_t.at[sidx.at[0]], bV, sb).wait()

    def _compute(bu):
        aV, bV, cV, tV = bu[0], bu[1], bu[2], bu[3]

        @plsc.parallel_loop(0, CH, step=1, unroll=4)
        def _(e):
            v0 = aV[e, pl.ds(0, 16)] + bV[e, pl.ds(0, 16)] + cV[pl.ds(H * e, 16)]
            tV[e, pl.ds(0, 16)] = jnp.maximum(v0, 0.0)
            v1 = (aV[e, pl.ds(16, 16)] + bV[e, pl.ds(16, 16)]
                  + cV[pl.ds(H * e + 16, 16)])
            tV[e, pl.ds(16, 16)] = jnp.maximum(v1, 0.0)

    def _scatter(g, bu):
        tV, ss = bu[3], bu[7]
        pltpu.async_copy(tV, s_sh.at[didx.at[g]], ss, add=True)

    def _wait_scatter(bu):
        tV, ss = bu[3], bu[7]
        pltpu.make_async_copy(tV, s_sh.at[didx.at[0]], ss).wait()

    _issue(0, bufs[0])

    def _step(k, c):
        g0 = 2 * k
        _issue(g0 + 1, bufs[1])
        _wait_loads(bufs[0])

        @pl.when(k > 0)
        def _():
            _wait_scatter(bufs[0])

        _compute(bufs[0])
        _scatter(g0, bufs[0])

        @pl.when(k < CPW // 2 - 1)
        def _():
            _issue(g0 + 2, bufs[0])

        _wait_loads(bufs[1])

        @pl.when(k > 0)
        def _():
            _wait_scatter(bufs[1])

        _compute(bufs[1])
        _scatter(g0 + 1, bufs[1])
        return c

    lax.fori_loop(0, CPW // 2, _step, 0)
    _wait_scatter(bufs[0])
    _wait_scatter(bufs[1])

    plsc.subcore_barrier()
    pltpu.sync_copy(s_sh.at[pl.ds(r0, RPW)], out_h.at[cid, pl.ds(r0, RPW)])


@functools.cache
def _sc_pass():
    return pl.kernel(
        _sc_body,
        out_type=jax.ShapeDtypeStruct((NC, NP, W), jnp.float32),
        mesh=plsc.VectorSubcoreMesh(core_axis_name="c", subcore_axis_name="s"),
        compiler_params=pltpu.CompilerParams(use_tc_tiling_on_sc=False),
        scratch_types=(
            [pltpu.VMEM((CPW, CH), jnp.int32)] * 2
            + [pltpu.VMEM((CH, H), jnp.float32)] * 4
            + [pltpu.VMEM((CH * H,), jnp.float32)] * 2
            + [pltpu.VMEM((CH, W), jnp.float32)] * 2
            + [pltpu.VMEM((ZR, W), jnp.float32)]
            + [pltpu.VMEM_SHARED((NP, W), jnp.float32)]
            + [pltpu.SemaphoreType.DMA] * 8
        ),
    )


def kernel(x, edge_attr, edge_index, batch,
           node_W1, node_b1, node_W2, node_b2,
           edge_W1, edge_b1, edge_W2, edge_b2,
           c1_W1, c1_b1, c1_W2, c1_b2,
           c2_W1, c2_b1, c2_W2, c2_b2):
    f32 = jnp.float32
    x_p = jnp.pad(x, ((0, NP - N), (0, 0)))
    batch3d = jnp.pad(batch, (0, NP - N)).reshape(NP // _BN, 1, _BN)
    src2 = jnp.pad(edge_index[0], (0, EP - E),
                   constant_values=NP - 1).reshape(EP // CH, CH)
    dst2 = jnp.pad(edge_index[1], (0, EP - E),
                   constant_values=NP - 1).reshape(EP // CH, CH)

    def row(b):
        return b.reshape(1, -1).astype(f32)

    def w2ext(w2, b2, cols):
        return jnp.concatenate(
            [w2, b2.reshape(1, cols), jnp.zeros((W - H - 1, cols), f32)], axis=0)

    w2e1 = w2ext(c1_W2, c1_b2, H)
    w2e2 = w2ext(c2_W2, c2_b2, OUT)

    h, a1, b1t = _tc_node(x_p, node_W1, row(node_b1), node_W2, row(node_b2),
                          c1_W1[0:H], c1_W1[H:2 * H], row(c1_b1))
    attr4 = edge_attr.reshape(E // 4, 4 * D_EDGE)
    eye4 = jnp.eye(4, dtype=f32)
    m1 = edge_W2 @ c1_W1[2 * H:3 * H]
    m2 = edge_W2 @ c2_W1[2 * H:3 * H]
    r1 = edge_b2 @ c1_W1[2 * H:3 * H]
    r2 = edge_b2 @ c2_W1[2 * H:3 * H]

    def bd(w):
        return jnp.kron(eye4, w)

    def row4(b):
        return jnp.tile(b.reshape(1, -1), (1, 4)).astype(f32)

    c1t, c2t = _tc_edge(attr4, bd(edge_W1), row4(edge_b1),
                        bd(m1), row4(r1), bd(m2), row4(r2))

    sc = _sc_pass()
    sp1 = sc(a1, b1t, c1t.reshape(-1), src2, dst2)
    a2, b2t = _tc_update(sp1, h, w2e1, c2_W1[0:H], c2_W1[H:2 * H], row(c2_b1))
    sp2 = sc(a2, b2t, c2t.reshape(-1), src2, dst2)
    return _tc_pool(sp2, w2e2, batch3d)


# speedup vs baseline: 1.2145x; 1.0560x over previous
import functools
import jax
import jax.numpy as jnp
from jax import lax
from jax.experimental import pallas as pl
from jax.experimental.pallas import tpu as pltpu
from jax.experimental.pallas import tpu_sc as plsc

N, E, D_FEAT, D_EDGE, H, OUT, G = 10000, 320000, 128, 16, 32, 32, 64
NP = 10240
W = 48
NC, NS = 2, 16
NWORK = NC * NS
CH = 128
EP = 327680
NCHUNKS = EP // CH
CPW0, CPW1 = 104, 56
RPW = NP // NS
ZR = 160


def _node_body(x, nw1, nb1, nw2, nb2, w1d, w1s, cb1, h_o, a_o, b_o):
    hid = jnp.maximum(jnp.dot(x[...], nw1[...],
                              preferred_element_type=jnp.float32) + nb1[...], 0.0)
    h = jnp.dot(hid, nw2[...], preferred_element_type=jnp.float32) + nb2[...]
    h_o[...] = h
    a_o[...] = jnp.dot(h, w1d[...], preferred_element_type=jnp.float32) + cb1[...]
    b_o[...] = jnp.dot(h, w1s[...], preferred_element_type=jnp.float32)


def _edge_body(attr4, ew1b, eb1b, m1b, r1b, m2b, r2b, c1_o, c2_o):
    hid = jnp.maximum(jnp.dot(attr4[...], ew1b[...],
                              preferred_element_type=jnp.float32) + eb1b[...], 0.0)
    c1_o[...] = jnp.dot(hid, m1b[...], preferred_element_type=jnp.float32) + r1b[...]
    c2_o[...] = jnp.dot(hid, m2b[...], preferred_element_type=jnp.float32) + r2b[...]


def _update_body(sp, h, w2e, w1d, w1s, cb1, a_o, b_o):
    s = sp[0] + sp[1]
    agg = jnp.dot(s, w2e[...], preferred_element_type=jnp.float32)
    h2 = jnp.maximum(h[...] + agg, 0.0)
    a_o[...] = jnp.dot(h2, w1d[...], preferred_element_type=jnp.float32) + cb1[...]
    b_o[...] = jnp.dot(h2, w1s[...], preferred_element_type=jnp.float32)


def _pool_body(sp, w2e, batch, out):
    i = pl.program_id(0)
    s = sp[0] + sp[1]
    node_out = jnp.dot(s, w2e[...], preferred_element_type=jnp.float32)
    rows = s.shape[0]
    rid = lax.broadcasted_iota(jnp.int32, (rows, 1), 0) + i * rows
    node_out = jnp.where(rid < N, node_out, 0.0)
    b_ids = batch[0]
    gid = lax.broadcasted_iota(jnp.int32, (G, rows), 0)
    col = lax.broadcasted_iota(jnp.int32, (G, rows), 1) + i * rows
    oh = jnp.where((gid == b_ids) & (col < N), 1.0, 0.0).astype(jnp.float32)
    acc = jnp.dot(oh, node_out, preferred_element_type=jnp.float32)

    @pl.when(i == 0)
    def _():
        out[...] = jnp.zeros_like(out)

    out[...] += acc


_BN = 512
_BE8 = 512


def _full(shape):
    return pl.BlockSpec(shape, lambda i: tuple(0 for _ in shape))


def _tc_node(x_p, nw1, nb1, nw2, nb2, w1d, w1s, cb1):
    grid = NP // _BN
    outs = [jax.ShapeDtypeStruct((NP, H), jnp.float32)] * 3
    return pl.pallas_call(
        _node_body,
        grid=(grid,),
        in_specs=[
            pl.BlockSpec((_BN, D_FEAT), lambda i: (i, 0)),
            _full((D_FEAT, H)), _full((1, H)), _full((H, H)), _full((1, H)),
            _full((H, H)), _full((H, H)), _full((1, H)),
        ],
        out_specs=[pl.BlockSpec((_BN, H), lambda i: (i, 0))] * 3,
        out_shape=outs,
    )(x_p, nw1, nb1, nw2, nb2, w1d, w1s, cb1)


def _tc_edge(attr8, ew1b, eb1b, m1b, r1b, m2b, r2b):
    grid = (EP // 8) // _BE8
    outs = [jax.ShapeDtypeStruct((EP // 8, 8 * H), jnp.float32)] * 2
    return pl.pallas_call(
        _edge_body,
        grid=(grid,),
        in_specs=[
            pl.BlockSpec((_BE8, 8 * D_EDGE),
                         lambda i: (jnp.minimum(i, (E // 8) // _BE8), 0)),
            _full((8 * D_EDGE, 8 * H)), _full((1, 8 * H)),
            _full((8 * H, 8 * H)), _full((1, 8 * H)),
            _full((8 * H, 8 * H)), _full((1, 8 * H)),
        ],
        out_specs=[pl.BlockSpec((_BE8, 8 * H), lambda i: (i, 0))] * 2,
        out_shape=outs,
    )(attr8, ew1b, eb1b, m1b, r1b, m2b, r2b)


def _tc_update(sp, h, w2e, w1d, w1s, cb1):
    grid = NP // _BN
    outs = [jax.ShapeDtypeStruct((NP, H), jnp.float32)] * 2
    return pl.pallas_call(
        _update_body,
        grid=(grid,),
        in_specs=[
            pl.BlockSpec((NC, _BN, W), lambda i: (0, i, 0)),
            pl.BlockSpec((_BN, H), lambda i: (i, 0)),
            _full((W, H)), _full((H, H)), _full((H, H)), _full((1, H)),
        ],
        out_specs=[pl.BlockSpec((_BN, H), lambda i: (i, 0))] * 2,
        out_shape=outs,
    )(sp, h, w2e, w1d, w1s, cb1)


def _tc_pool(sp, w2e, batch3d):
    grid = NP // _BN
    return pl.pallas_call(
        _pool_body,
        grid=(grid,),
        in_specs=[
            pl.BlockSpec((NC, _BN, W), lambda i: (0, i, 0)),
            _full((W, OUT)),
            pl.BlockSpec((1, 1, _BN), lambda i: (i, 0, 0)),
        ],
        out_specs=pl.BlockSpec((G, OUT), lambda i: (0, 0)),
        out_shape=jax.ShapeDtypeStruct((G, OUT), jnp.float32),
    )(sp, w2e, batch3d)


def _sc_body(a_t, b_t, c4, src2, dst2, out_h,
             sidx, didx, a0, a1, b0, b1, c0, c1, t0, t1, z_v, s_sh,
             sa0, sa1, sb0, sb1, sc0, sc1, ss0, ss1):
    cid = lax.axis_index("c")
    sid = lax.axis_index("s")
    r0 = sid * RPW
    wrow0 = jnp.where(cid == 0, sid * CPW0, NS * CPW0 + sid * CPW1)
    nsteps = jnp.where(cid == 0, CPW0 // 2, CPW1 // 2)

    zv = jnp.zeros((16,), jnp.float32)

    def _zrow(i, c):
        z_v[i, pl.ds(0, 16)] = zv
        z_v[i, pl.ds(16, 16)] = zv
        z_v[i, pl.ds(32, 16)] = zv
        return c

    lax.fori_loop(0, ZR, _zrow, 0)

    onev = jnp.where(lax.iota(jnp.int32, 16) == 0,
                     jnp.float32(1.0), jnp.float32(0.0))

    def _trow(i, c):
        t0[i, pl.ds(32, 16)] = onev
        t1[i, pl.ds(32, 16)] = onev
        return c

    lax.fori_loop(0, CH, _trow, 0)

    pltpu.sync_copy(src2.at[pl.ds(wrow0, CPW0)], sidx)
    pltpu.sync_copy(dst2.at[pl.ds(wrow0, CPW0)], didx)

    for q in range(RPW // ZR):
        pltpu.sync_copy(z_v, s_sh.at[pl.ds(r0 + q * ZR, ZR)])
    plsc.subcore_barrier()

    bufs = ((a0, b0, c0, t0, sa0, sb0, sc0, ss0),
            (a1, b1, c1, t1, sa1, sb1, sc1, ss1))

    def _issue(g, bu):
        aV, bV, cV, _, sa, sb, sc, _ = bu
        pltpu.async_copy(c4.at[pl.ds((wrow0 + g) * (CH * H), CH * H)], cV, sc)
        pltpu.async_copy(a_t.at[didx.at[g]], aV, sa)
        pltpu.async_copy(b_t.at[sidx.at[g]], bV, sb)

    def _wait_loads(bu):
        aV, bV, cV, _, sa, sb, sc, _ = bu
        pltpu.make_async_copy(c4.at[pl.ds(0, CH * H)], cV, sc).wait()
        pltpu.make_async_copy(a_t.at[didx.at[0]], aV, sa).wait()
        pltpu.make_async_copy(b_t.at[sidx.at[0]], bV, sb).wait()

    def _compute(bu):
        aV, bV, cV, tV = bu[0], bu[1], bu[2], bu[3]

        @plsc.parallel_loop(0, CH, step=1, unroll=4)
        def _(e):
            v0 = aV[e, pl.ds(0, 16)] + bV[e, pl.ds(0, 16)] + cV[pl.ds(H * e, 16)]
            tV[e, pl.ds(0, 16)] = jnp.maximum(v0, 0.0)
            v1 = (aV[e, pl.ds(16, 16)] + bV[e, pl.ds(16, 16)]
                  + cV[pl.ds(H * e + 16, 16)])
            tV[e, pl.ds(16, 16)] = jnp.maximum(v1, 0.0)

    def _scatter(g, bu):
        tV, ss = bu[3], bu[7]
        pltpu.async_copy(tV, s_sh.at[didx.at[g]], ss, add=True)

    def _wait_scatter(bu):
        tV, ss = bu[3], bu[7]
        pltpu.make_async_copy(tV, s_sh.at[didx.at[0]], ss).wait()

    _issue(0, bufs[0])

    def _step(k, c):
        g0 = 2 * k
        _issue(g0 + 1, bufs[1])
        _wait_loads(bufs[0])

        @pl.when(k > 0)
        def _():
            _wait_scatter(bufs[0])

        _compute(bufs[0])
        _scatter(g0, bufs[0])

        @pl.when(k < nsteps - 1)
        def _():
            _issue(g0 + 2, bufs[0])

        _wait_loads(bufs[1])

        @pl.when(k > 0)
        def _():
            _wait_scatter(bufs[1])

        _compute(bufs[1])
        _scatter(g0 + 1, bufs[1])
        return c

    lax.fori_loop(0, nsteps, _step, 0)
    _wait_scatter(bufs[0])
    _wait_scatter(bufs[1])

    plsc.subcore_barrier()
    pltpu.sync_copy(s_sh.at[pl.ds(r0, RPW)], out_h.at[cid, pl.ds(r0, RPW)])


@functools.cache
def _sc_pass():
    return pl.kernel(
        _sc_body,
        out_type=jax.ShapeDtypeStruct((NC, NP, W), jnp.float32),
        mesh=plsc.VectorSubcoreMesh(core_axis_name="c", subcore_axis_name="s"),
        compiler_params=pltpu.CompilerParams(use_tc_tiling_on_sc=False),
        scratch_types=(
            [pltpu.VMEM((CPW0, CH), jnp.int32)] * 2
            + [pltpu.VMEM((CH, H), jnp.float32)] * 4
            + [pltpu.VMEM((CH * H,), jnp.float32)] * 2
            + [pltpu.VMEM((CH, W), jnp.float32)] * 2
            + [pltpu.VMEM((ZR, W), jnp.float32)]
            + [pltpu.VMEM_SHARED((NP, W), jnp.float32)]
            + [pltpu.SemaphoreType.DMA] * 8
        ),
    )


def kernel(x, edge_attr, edge_index, batch,
           node_W1, node_b1, node_W2, node_b2,
           edge_W1, edge_b1, edge_W2, edge_b2,
           c1_W1, c1_b1, c1_W2, c1_b2,
           c2_W1, c2_b1, c2_W2, c2_b2):
    f32 = jnp.float32
    x_p = jnp.pad(x, ((0, NP - N), (0, 0)))
    batch3d = jnp.pad(batch, (0, NP - N)).reshape(NP // _BN, 1, _BN)
    idx_rows = NS * CPW0 + (NS - 1) * CPW1 + CPW0
    src2 = jnp.pad(jnp.pad(edge_index[0], (0, EP - E),
                           constant_values=NP - 1).reshape(EP // CH, CH),
                   ((0, idx_rows - EP // CH), (0, 0)), constant_values=NP - 1)
    dst2 = jnp.pad(jnp.pad(edge_index[1], (0, EP - E),
                           constant_values=NP - 1).reshape(EP // CH, CH),
                   ((0, idx_rows - EP // CH), (0, 0)), constant_values=NP - 1)

    def row(b):
        return b.reshape(1, -1).astype(f32)

    def w2ext(w2, b2, cols):
        return jnp.concatenate(
            [w2, b2.reshape(1, cols), jnp.zeros((W - H - 1, cols), f32)], axis=0)

    w2e1 = w2ext(c1_W2, c1_b2, H)
    w2e2 = w2ext(c2_W2, c2_b2, OUT)

    h, a1, b1t = _tc_node(x_p, node_W1, row(node_b1), node_W2, row(node_b2),
                          c1_W1[0:H], c1_W1[H:2 * H], row(c1_b1))
    attr8 = edge_attr.reshape(E // 8, 8 * D_EDGE)
    eye8 = jnp.eye(8, dtype=f32)
    m1 = edge_W2 @ c1_W1[2 * H:3 * H]
    m2 = edge_W2 @ c2_W1[2 * H:3 * H]
    r1 = edge_b2 @ c1_W1[2 * H:3 * H]
    r2 = edge_b2 @ c2_W1[2 * H:3 * H]

    def bd(w):
        return jnp.kron(eye8, w)

    def row8(b):
        return jnp.tile(b.reshape(1, -1), (1, 8)).astype(f32)

    c1t, c2t = _tc_edge(attr8, bd(edge_W1), row8(edge_b1),
                        bd(m1), row8(r1), bd(m2), row8(r2))

    sc = _sc_pass()
    sp1 = sc(a1, b1t, c1t.reshape(-1), src2, dst2)
    a2, b2t = _tc_update(sp1, h, w2e1, c2_W1[0:H], c2_W1[H:2 * H], row(c2_b1))
    sp2 = sc(a2, b2t, c2t.reshape(-1), src2, dst2)
    return _tc_pool(sp2, w2e2, batch3d)


# speedup vs baseline: 1.2150x; 1.0004x over previous
import functools
import jax
import jax.numpy as jnp
from jax import lax
from jax.experimental import pallas as pl
from jax.experimental.pallas import tpu as pltpu
from jax.experimental.pallas import tpu_sc as plsc

N, E, D_FEAT, D_EDGE, H, OUT, G = 10000, 320000, 128, 16, 32, 32, 64
NP = 10240
W = 48
NC, NS = 2, 16
CH = 128
EP = 327680
CPW0, CPW1 = 104, 56
RPW = NP // NS
ZR = 160


def _node_body(x, nw1, nb1, nw2, nb2, w1d, w1s, cb1, h_o, a_o, b_o):
    hid = jnp.maximum(jnp.dot(x[...], nw1[...],
                              preferred_element_type=jnp.float32) + nb1[...], 0.0)
    h = jnp.dot(hid, nw2[...], preferred_element_type=jnp.float32) + nb2[...]
    h_o[...] = h
    a_o[...] = jnp.dot(h, w1d[...], preferred_element_type=jnp.float32) + cb1[...]
    b_o[...] = jnp.dot(h, w1s[...], preferred_element_type=jnp.float32)


def _edge_body(attr4, ew1b, eb1b, m1b, r1b, m2b, r2b, c1_o, c2_o):
    hid = jnp.maximum(jnp.dot(attr4[...], ew1b[...],
                              preferred_element_type=jnp.float32) + eb1b[...], 0.0)
    c1_o[...] = jnp.dot(hid, m1b[...], preferred_element_type=jnp.float32) + r1b[...]
    c2_o[...] = jnp.dot(hid, m2b[...], preferred_element_type=jnp.float32) + r2b[...]


def _update_body(sp, h, w2e, w1d, w1s, cb1, a_o, b_o):
    s = sp[0] + sp[1]
    agg = jnp.dot(s, w2e[...], preferred_element_type=jnp.float32)
    h2 = jnp.maximum(h[...] + agg, 0.0)
    a_o[...] = jnp.dot(h2, w1d[...], preferred_element_type=jnp.float32) + cb1[...]
    b_o[...] = jnp.dot(h2, w1s[...], preferred_element_type=jnp.float32)


def _pool_body(sp, w2e, batch, out):
    i = pl.program_id(0)
    s = sp[0] + sp[1]
    node_out = jnp.dot(s, w2e[...], preferred_element_type=jnp.float32)
    rows = s.shape[0]
    rid = lax.broadcasted_iota(jnp.int32, (rows, 1), 0) + i * rows
    node_out = jnp.where(rid < N, node_out, 0.0)
    b_ids = batch[0]
    gid = lax.broadcasted_iota(jnp.int32, (G, rows), 0)
    col = lax.broadcasted_iota(jnp.int32, (G, rows), 1) + i * rows
    oh = jnp.where((gid == b_ids) & (col < N), 1.0, 0.0).astype(jnp.float32)
    acc = jnp.dot(oh, node_out, preferred_element_type=jnp.float32)

    @pl.when(i == 0)
    def _():
        out[...] = jnp.zeros_like(out)

    out[...] += acc


_BN = 512
_BE8 = 512


def _full(shape):
    return pl.BlockSpec(shape, lambda i: tuple(0 for _ in shape))


def _tc_node(x_p, nw1, nb1, nw2, nb2, w1d, w1s, cb1):
    grid = NP // _BN
    outs = [jax.ShapeDtypeStruct((NP, H), jnp.float32)] * 3
    return pl.pallas_call(
        _node_body,
        grid=(grid,),
        in_specs=[
            pl.BlockSpec((_BN, D_FEAT), lambda i: (i, 0)),
            _full((D_FEAT, H)), _full((1, H)), _full((H, H)), _full((1, H)),
            _full((H, H)), _full((H, H)), _full((1, H)),
        ],
        out_specs=[pl.BlockSpec((_BN, H), lambda i: (i, 0))] * 3,
        out_shape=outs,
    )(x_p, nw1, nb1, nw2, nb2, w1d, w1s, cb1)


def _tc_edge(attr8, ew1b, eb1b, m1b, r1b, m2b, r2b):
    grid = (EP // 8) // _BE8
    outs = [jax.ShapeDtypeStruct((EP // 8, 8 * H), jnp.float32)] * 2
    return pl.pallas_call(
        _edge_body,
        grid=(grid,),
        in_specs=[
            pl.BlockSpec((_BE8, 8 * D_EDGE),
                         lambda i: (jnp.minimum(i, (E // 8) // _BE8), 0)),
            _full((8 * D_EDGE, 8 * H)), _full((1, 8 * H)),
            _full((8 * H, 8 * H)), _full((1, 8 * H)),
            _full((8 * H, 8 * H)), _full((1, 8 * H)),
        ],
        out_specs=[pl.BlockSpec((_BE8, 8 * H), lambda i: (i, 0))] * 2,
        out_shape=outs,
    )(attr8, ew1b, eb1b, m1b, r1b, m2b, r2b)


def _tc_update(sp, h, w2e, w1d, w1s, cb1):
    grid = NP // _BN
    outs = [jax.ShapeDtypeStruct((NP, H), jnp.float32)] * 2
    return pl.pallas_call(
        _update_body,
        grid=(grid,),
        in_specs=[
            pl.BlockSpec((NC, _BN, W), lambda i: (0, i, 0)),
            pl.BlockSpec((_BN, H), lambda i: (i, 0)),
            _full((W, H)), _full((H, H)), _full((H, H)), _full((1, H)),
        ],
        out_specs=[pl.BlockSpec((_BN, H), lambda i: (i, 0))] * 2,
        out_shape=outs,
    )(sp, h, w2e, w1d, w1s, cb1)


def _tc_pool(sp, w2e, batch3d):
    grid = NP // _BN
    return pl.pallas_call(
        _pool_body,
        grid=(grid,),
        in_specs=[
            pl.BlockSpec((NC, _BN, W), lambda i: (0, i, 0)),
            _full((W, OUT)),
            pl.BlockSpec((1, 1, _BN), lambda i: (i, 0, 0)),
        ],
        out_specs=pl.BlockSpec((G, OUT), lambda i: (0, 0)),
        out_shape=jax.ShapeDtypeStruct((G, OUT), jnp.float32),
    )(sp, w2e, batch3d)


def _sc_body(a_t, b_t, c4, src2, dst2, out_h,
             sidx, didx, a0, a1, b0, b1, c0, c1, t0, t1, z_v, s_sh,
             sa0, sa1, sb0, sb1, sc0, sc1, ss0, ss1):
    cid = lax.axis_index("c")
    sid = lax.axis_index("s")
    r0 = sid * RPW
    wrow0 = jnp.where(cid == 0, sid * CPW0, NS * CPW0 + sid * CPW1)
    nsteps = jnp.where(cid == 0, CPW0 // 2, CPW1 // 2)

    zv = jnp.zeros((16,), jnp.float32)

    def _zrow(i, c):
        z_v[i, pl.ds(0, 16)] = zv
        z_v[i, pl.ds(16, 16)] = zv
        z_v[i, pl.ds(32, 16)] = zv
        return c

    lax.fori_loop(0, ZR, _zrow, 0)

    onev = jnp.where(lax.iota(jnp.int32, 16) == 0,
                     jnp.float32(1.0), jnp.float32(0.0))

    def _trow(i, c):
        t0[i, pl.ds(32, 16)] = onev
        t1[i, pl.ds(32, 16)] = onev
        return c

    lax.fori_loop(0, CH, _trow, 0)

    pltpu.sync_copy(src2.at[pl.ds(wrow0, CPW0)], sidx)
    pltpu.sync_copy(dst2.at[pl.ds(wrow0, CPW0)], didx)

    for q in range(RPW // ZR):
        pltpu.sync_copy(z_v, s_sh.at[pl.ds(r0 + q * ZR, ZR)])
    plsc.subcore_barrier()

    bufs = ((a0, b0, c0, t0, sa0, sb0, sc0, ss0),
            (a1, b1, c1, t1, sa1, sb1, sc1, ss1))

    def _issue(g, bu):
        aV, bV, cV, _, sa, sb, sc, _ = bu
        pltpu.async_copy(c4.at[pl.ds((wrow0 + g) * (CH * H), CH * H)], cV, sc)
        pltpu.async_copy(a_t.at[didx.at[g]], aV, sa)
        pltpu.async_copy(b_t.at[sidx.at[g]], bV, sb)

    def _wait_loads(bu):
        aV, bV, cV, _, sa, sb, sc, _ = bu
        pltpu.make_async_copy(c4.at[pl.ds(0, CH * H)], cV, sc).wait()
        pltpu.make_async_copy(a_t.at[didx.at[0]], aV, sa).wait()
        pltpu.make_async_copy(b_t.at[sidx.at[0]], bV, sb).wait()

    def _compute(bu):
        aV, bV, cV, tV = bu[0], bu[1], bu[2], bu[3]

        @plsc.parallel_loop(0, CH, step=1, unroll=4)
        def _(e):
            v0 = aV[e, pl.ds(0, 16)] + bV[e, pl.ds(0, 16)] + cV[pl.ds(H * e, 16)]
            tV[e, pl.ds(0, 16)] = jnp.maximum(v0, 0.0)
            v1 = (aV[e, pl.ds(16, 16)] + bV[e, pl.ds(16, 16)]
                  + cV[pl.ds(H * e + 16, 16)])
            tV[e, pl.ds(16, 16)] = jnp.maximum(v1, 0.0)

    def _scatter(g, bu):
        tV, ss = bu[3], bu[7]
        pltpu.async_copy(tV, s_sh.at[didx.at[g]], ss, add=True)

    def _wait_scatter(bu):
        tV, ss = bu[3], bu[7]
        pltpu.make_async_copy(tV, s_sh.at[didx.at[0]], ss).wait()

    _issue(0, bufs[0])

    def _step(k, c):
        g0 = 2 * k
        _issue(g0 + 1, bufs[1])
        _wait_loads(bufs[0])

        @pl.when(k > 0)
        def _():
            _wait_scatter(bufs[0])

        _compute(bufs[0])
        _scatter(g0, bufs[0])

        @pl.when(k < nsteps - 1)
        def _():
            _issue(g0 + 2, bufs[0])

        _wait_loads(bufs[1])

        @pl.when(k > 0)
        def _():
            _wait_scatter(bufs[1])

        _compute(bufs[1])
        _scatter(g0 + 1, bufs[1])
        return c

    lax.fori_loop(0, nsteps, _step, 0)
    _wait_scatter(bufs[0])
    _wait_scatter(bufs[1])

    plsc.subcore_barrier()
    pltpu.sync_copy(s_sh.at[pl.ds(r0, RPW)], out_h.at[cid, pl.ds(r0, RPW)])


@functools.cache
def _sc_pass():
    return pl.kernel(
        _sc_body,
        out_type=jax.ShapeDtypeStruct((NC, NP, W), jnp.float32),
        mesh=plsc.VectorSubcoreMesh(core_axis_name="c", subcore_axis_name="s"),
        compiler_params=pltpu.CompilerParams(use_tc_tiling_on_sc=False),
        scratch_types=(
            [pltpu.VMEM((CPW0, CH), jnp.int32)] * 2
            + [pltpu.VMEM((CH, H), jnp.float32)] * 4
            + [pltpu.VMEM((CH * H,), jnp.float32)] * 2
            + [pltpu.VMEM((CH, W), jnp.float32)] * 2
            + [pltpu.VMEM((ZR, W), jnp.float32)]
            + [pltpu.VMEM_SHARED((NP, W), jnp.float32)]
            + [pltpu.SemaphoreType.DMA] * 8
        ),
    )


def kernel(x, edge_attr, edge_index, batch,
           node_W1, node_b1, node_W2, node_b2,
           edge_W1, edge_b1, edge_W2, edge_b2,
           c1_W1, c1_b1, c1_W2, c1_b2,
           c2_W1, c2_b1, c2_W2, c2_b2):
    f32 = jnp.float32
    x_p = jnp.pad(x, ((0, NP - N), (0, 0)))
    batch3d = jnp.pad(batch, (0, NP - N)).reshape(NP // _BN, 1, _BN)
    idx_rows = NS * CPW0 + (NS - 1) * CPW1 + CPW0
    src2 = jnp.pad(jnp.pad(edge_index[0], (0, EP - E),
                           constant_values=NP - 1).reshape(EP // CH, CH),
                   ((0, idx_rows - EP // CH), (0, 0)), constant_values=NP - 1)
    dst2 = jnp.pad(jnp.pad(edge_index[1], (0, EP - E),
                           constant_values=NP - 1).reshape(EP // CH, CH),
                   ((0, idx_rows - EP // CH), (0, 0)), constant_values=NP - 1)

    def row(b):
        return b.reshape(1, -1).astype(f32)

    def w2ext(w2, b2, cols):
        return jnp.concatenate(
            [w2, b2.reshape(1, cols), jnp.zeros((W - H - 1, cols), f32)], axis=0)

    w2e1 = w2ext(c1_W2, c1_b2, H)
    w2e2 = w2ext(c2_W2, c2_b2, OUT)

    h, a1, b1t = _tc_node(x_p, node_W1, row(node_b1), node_W2, row(node_b2),
                          c1_W1[0:H], c1_W1[H:2 * H], row(c1_b1))
    attr8 = edge_attr.reshape(E // 8, 8 * D_EDGE)
    eye8 = jnp.eye(8, dtype=f32)
    m1 = edge_W2 @ c1_W1[2 * H:3 * H]
    m2 = edge_W2 @ c2_W1[2 * H:3 * H]
    r1 = edge_b2 @ c1_W1[2 * H:3 * H]
    r2 = edge_b2 @ c2_W1[2 * H:3 * H]

    def bd(w):
        return jnp.kron(eye8, w)

    def row8(b):
        return jnp.tile(b.reshape(1, -1), (1, 8)).astype(f32)

    c1t, c2t = _tc_edge(attr8, bd(edge_W1), row8(edge_b1),
                        bd(m1), row8(r1), bd(m2), row8(r2))

    sc = _sc_pass()
    sp1 = sc(a1, b1t, c1t.reshape(-1), src2, dst2)
    a2, b2t = _tc_update(sp1, h, w2e1, c2_W1[0:H], c2_W1[H:2 * H], row(c2_b1))
    sp2 = sc(a2, b2t, c2t.reshape(-1), src2, dst2)
    return _tc_pool(sp2, w2e2, batch3d)
